# baseline (speedup 1.0000x reference)
import functools
import math

import jax
import jax.numpy as jnp
from jax import lax
from jax.experimental import pallas as pl
from jax.experimental.pallas import tpu as pltpu
from jax.experimental.pallas import tpu_sc as plsc

_NC = 2
_NS = 16
_NW = _NC * _NS


def _silu(v):
    return v * jax.nn.sigmoid(v)


def _node_body(x_ref, wi1a_ref, wi1c_ref, bi1_ref, wm1_ref, bm1_ref,
               wm2_ref, bm2_ref, m_ref, a_ref, c_ref, g_ref, mt_ref):
    x = x_ref[...]
    a_ref[...] = (jnp.dot(x, wi1a_ref[...], preferred_element_type=jnp.float32)
                  + bi1_ref[...][None, :])
    c_ref[...] = jnp.dot(x, wi1c_ref[...], preferred_element_type=jnp.float32)
    t = _silu(jnp.dot(x, wm1_ref[...], preferred_element_type=jnp.float32)
              + bm1_ref[...][None, :])
    g = _silu(jnp.dot(t, wm2_ref[...], preferred_element_type=jnp.float32)
              + bm2_ref[...][None, :])
    g_ref[...] = g
    mt_ref[...] = jnp.dot(g, m_ref[...], preferred_element_type=jnp.float32)


def _edge_body(p_ref, ies_ref, nd_ref, wi1b_ref, wi2_ref, bi2_ref, out_ref):
    bf = jnp.bfloat16
    pre = p_ref[...] + jnp.dot(ies_ref[...].astype(bf), wi1b_ref[...].astype(bf),
                               preferred_element_type=jnp.float32)
    h = jnp.dot(_silu(pre).astype(bf), wi2_ref[...].astype(bf),
                preferred_element_type=jnp.float32)
    h = h + bi2_ref[...][None, :]
    decay = jnp.cos((math.pi / 2.0) * nd_ref[0, 0, :])
    out_ref[...] = decay[:, None] * _silu(h)


def _final_body(CH, *refs):
    ps = refs[0:CH]
    pc = refs[CH:2 * CH]
    (mt_ref, wu1_ref, bu1_ref, wu2_ref, bu2_ref,
     gamma_ref, beta_ref, out_ref) = refs[2 * CH:]
    sums = ps[0][0] + ps[0][1]
    cnt = pc[0][0, :, 0:1] + pc[0][1, :, 0:1]
    for r in ps[1:]:
        sums = sums + r[0] + r[1]
    for r in pc[1:]:
        cnt = cnt + r[0, :, 0:1] + r[1, :, 0:1]
    incoming = sums / jnp.maximum(cnt, 1.0)
    z = mt_ref[...] + incoming
    h = _silu(jnp.dot(z, wu1_ref[...], preferred_element_type=jnp.float32)
              + bu1_ref[...][None, :])
    h = (jnp.dot(h, wu2_ref[...], preferred_element_type=jnp.float32)
         + bu2_ref[...][None, :])
    mean = jnp.mean(h, axis=0, keepdims=True)
    var = jnp.mean((h - mean) ** 2, axis=0, keepdims=True)
    h = (h - mean) / jnp.sqrt(var + 1e-5) * gamma_ref[...][None, :] \
        + beta_ref[...][None, :]
    out_ref[...] = _silu(h)


def _sc_gather_body(K, NB, EPW, NPT, CO, a_hbm, c_hbm, src_hbm, dst_hbm,
                    p_hbm, cnt_out, cnt_sh,
                    idx_s0, idx_d0, idx_s1, idx_d1, ra0, rc0, ra1, rc1, rw,
                    zcnt, ones_v,
                    sis0, sid0, sis1, sid1, sga0, sgc0, sga1, sgc1, swr):
    cid = lax.axis_index("c")
    sid = lax.axis_index("s")
    wid = sid * _NC + cid
    base = wid * EPW
    row0 = sid * NPT
    nch = ra0.shape[1] // 16
    zero16 = jnp.zeros((16,), jnp.float32)
    one16 = jnp.ones((16,), jnp.float32)

    def zcfill(i, carry):
        zcnt[i, :] = zero16
        return carry

    lax.fori_loop(0, NPT, zcfill, 0)

    def onesfill(i, carry):
        ones_v[i, :] = one16
        return carry

    lax.fori_loop(0, K, onesfill, 0)

    pltpu.sync_copy(zcnt, cnt_sh.at[pl.ds(row0, NPT)])
    plsc.subcore_barrier()

    I = [(idx_s0, idx_d0, sis0, sid0), (idx_s1, idx_d1, sis1, sid1)]
    S = [(ra0, rc0, sga0, sgc0), (ra1, rc1, sga1, sgc1)]

    def issue_idx(jn, p):
        ibs, ibd, ss, sd = I[p]
        off = CO + base + jn * K
        pltpu.async_copy(src_hbm.at[pl.ds(off, K)], ibs, ss)
        pltpu.async_copy(dst_hbm.at[pl.ds(off, K)], ibd, sd)

    def wait_idx(p):
        ibs, ibd, ss, sd = I[p]
        pltpu.make_async_copy(src_hbm.at[pl.ds(0, K)], ibs, ss).wait()
        pltpu.make_async_copy(dst_hbm.at[pl.ds(0, K)], ibd, sd).wait()

    def issue_gathers(p):
        ibs, ibd = I[p][0], I[p][1]
        ra, rc, sga, sgc = S[p]
        pltpu.async_copy(a_hbm.at[ibs], ra, sga)
        pltpu.async_copy(c_hbm.at[ibd], rc, sgc)

    def wait_gathers(p):
        ibs = I[p][0]
        ra, rc, sga, sgc = S[p]
        pltpu.make_async_copy(a_hbm.at[ibs], ra, sga).wait()
        pltpu.make_async_copy(c_hbm.at[ibs], rc, sgc).wait()

    def drain_write():
        pltpu.make_async_copy(rw, p_hbm.at[pl.ds(0, K)], swr).wait()

    def finish(j, p, first):
        ra, rc = S[p][0], S[p][1]
        if not first:
            drain_write()

        @plsc.parallel_loop(0, K, 1, unroll=4)
        def addrow(i):
            for kk in range(nch):
                sl = pl.ds(kk * 16, 16)
                rw[i, sl] = ra[i, sl] + rc[i, sl]

        pltpu.async_copy(rw, p_hbm.at[pl.ds(base + j * K, K)], swr)
        pltpu.sync_copy(ones_v, cnt_sh.at[I[p][1]], add=True)

    def body(j, p, first=False):
        wait_gathers(p)
        wait_idx(1 - p)
        issue_gathers(1 - p)
        finish(j, p, first)
        issue_idx(jnp.minimum(j + 2, NB - 1), p)

    issue_idx(0, 0)
    issue_idx(1, 1)
    wait_idx(0)
    issue_gathers(0)
    body(jnp.int32(0), 0, first=True)

    def pair(j2, carry):
        j = 2 * j2 + 1
        body(j, 1)
        body(j + 1, 0)
        return carry

    lax.fori_loop(0, (NB - 2) // 2, pair, 0)
    if (NB - 2) % 2 == 1:
        body(jnp.int32(NB - 2), 1)
    pe = (NB - 1) % 2
    wait_gathers(pe)
    finish(jnp.int32(NB - 1), pe, False)
    wait_idx(NB % 2)
    drain_write()
    plsc.subcore_barrier()
    pltpu.sync_copy(cnt_sh.at[pl.ds(row0, NPT)],
                    cnt_out.at[cid, pl.ds(row0, NPT)])


def _sc_scatter_body(K, NB, EPW, NPT, CO, H, imp_hbm, g_hbm, src_hbm, dst_hbm,
                     sums_out, sums_sh,
                     idx_s0, idx_d0, idx_s1, idx_d1, m_v0, g_v0, m_v1, g_v1,
                     zrow,
                     sis0, sid0, sis1, sid1, sga0, smv0, sga1, smv1):
    cid = lax.axis_index("c")
    sid = lax.axis_index("s")
    wid = sid * _NC + cid
    base = wid * EPW
    row0 = sid * NPT
    nch = H // 16
    zero16 = jnp.zeros((16,), jnp.float32)

    zr = NPT // 5

    def zfill(i, carry):
        for kk in range(nch):
            zrow[i, pl.ds(kk * 16, 16)] = zero16
        return carry

    lax.fori_loop(0, zr, zfill, 0)

    for q in range(5):
        pltpu.sync_copy(zrow, sums_sh.at[pl.ds(row0 + q * zr, zr)])
    plsc.subcore_barrier()

    I = [(idx_s0, idx_d0, sis0, sid0), (idx_s1, idx_d1, sis1, sid1)]
    S = [(m_v0, g_v0, sga0, smv0), (m_v1, g_v1, sga1, smv1)]

    def issue_idx(jn, p):
        ibs, ibd, ss, sd = I[p]
        off = CO + base + jn * K
        pltpu.async_copy(src_hbm.at[pl.ds(off, K)], ibs, ss)
        pltpu.async_copy(dst_hbm.at[pl.ds(off, K)], ibd, sd)

    def wait_idx(p):
        ibs, ibd, ss, sd = I[p]
        pltpu.make_async_copy(src_hbm.at[pl.ds(0, K)], ibs, ss).wait()
        pltpu.make_async_copy(dst_hbm.at[pl.ds(0, K)], ibd, sd).wait()

    def issue_gathers(j, p):
        ibs = I[p][0]
        mv, gv, sga, smv = S[p]
        pltpu.async_copy(g_hbm.at[ibs], gv, sga)
        pltpu.async_copy(imp_hbm.at[pl.ds(base + j * K, K)], mv, smv)

    def wait_gathers(p):
        ibs = I[p][0]
        mv, gv, sga, smv = S[p]
        pltpu.make_async_copy(g_hbm.at[ibs], gv, sga).wait()
        pltpu.make_async_copy(imp_hbm.at[pl.ds(0, K)], mv, smv).wait()

    def finish(p):
        mv, gv = S[p][0], S[p][1]

        @plsc.parallel_loop(0, K, 1, unroll=4)
        def mulrow(i):
            for kk in range(nch):
                sl = pl.ds(kk * 16, 16)
                mv[i, sl] = mv[i, sl] * gv[i, sl]

        pltpu.sync_copy(mv, sums_sh.at[I[p][1]], add=True)

    def body(j, p):
        wait_gathers(p)
        wait_idx(1 - p)
        issue_gathers(j + 1, 1 - p)
        finish(p)
        issue_idx(jnp.minimum(j + 2, NB - 1), p)

    issue_idx(0, 0)
    issue_idx(1, 1)
    wait_idx(0)
    issue_gathers(jnp.int32(0), 0)

    def pair(j2, carry):
        j = 2 * j2
        body(j, 0)
        body(j + 1, 1)
        return carry

    lax.fori_loop(0, (NB - 1) // 2, pair, 0)
    if (NB - 1) % 2 == 1:
        body(jnp.int32(NB - 2), 0)
    pe = (NB - 1) % 2
    wait_gathers(pe)
    finish(pe)
    wait_idx(NB % 2)
    plsc.subcore_barrier()
    pltpu.sync_copy(sums_sh.at[pl.ds(row0, NPT)],
                    sums_out.at[cid, pl.ds(row0, NPT)])


def kernel(x, edge_index, norm_distance, init_edge_states, Wi1, bi1, Wi2, bi2,
           Wm1, bm1, Wm2, bm2, Wu1, bu1, Wu2, bu2, gamma, beta, M):
    N, H = x.shape
    E = edge_index.shape[1]
    src = edge_index[0]
    dst = edge_index[1]
    Wi1a = Wi1[:H]
    Wi1b = Wi1[H:2 * H]
    Wi1c = Wi1[2 * H:]

    A, C, G, m_t = pl.pallas_call(
        _node_body,
        out_shape=[jax.ShapeDtypeStruct((N, H), jnp.float32)] * 4,
    )(x, Wi1a, Wi1c, bi1, Wm1, bm1, Wm2, bm2, M)

    CH = 2
    K = 80
    EPW_total = E // _NW
    NB_total = EPW_total // K
    assert EPW_total * _NW == E and NB_total * K == EPW_total
    assert K % 8 == 0 and K <= 128
    nb0 = NB_total // CH
    nbs = [nb0] * (CH - 1) + [NB_total - nb0 * (CH - 1)]
    NPT = 640
    N_pad = _NS * NPT
    assert N <= N_pad and NPT % 8 == 0 and NPT % 5 == 0
    mesh = plsc.VectorSubcoreMesh(core_axis_name="c", subcore_axis_name="s")
    sc_params = pltpu.CompilerParams(use_tc_tiling_on_sc=False)

    TE = 640
    assert (E // TE) * TE == E
    nd3 = norm_distance.reshape(E // TE, 1, TE)

    def sc_gather(co, NB):
        EPW = NB * K
        E_c = EPW * _NW
        return pl.kernel(
            functools.partial(_sc_gather_body, K, NB, EPW, NPT, co),
            out_type=[jax.ShapeDtypeStruct((E_c, H), jnp.float32),
                      jax.ShapeDtypeStruct((_NC, N_pad, 16), jnp.float32)],
            mesh=mesh,
            compiler_params=sc_params,
            scratch_types=(
                [pltpu.VMEM_SHARED((N_pad, 16), jnp.float32)]
                + [pltpu.VMEM((K,), jnp.int32)] * 4
                + [pltpu.VMEM((K, H), jnp.float32)] * 5
                + [pltpu.VMEM((NPT, 16), jnp.float32),
                   pltpu.VMEM((K, 16), jnp.float32)]
                + [pltpu.SemaphoreType.DMA] * 9
            ),
        )(A, C, src, dst)

    def tc_edge(co, NB, P_c):
        E_c = NB * K * _NW
        GE_c = E_c // TE
        assert GE_c * TE == E_c
        goff = co // TE
        return pl.pallas_call(
            _edge_body,
            grid=(GE_c,),
            in_specs=[
                pl.BlockSpec((TE, H), lambda i: (i, 0)),
                pl.BlockSpec((TE, H), lambda i: (i + goff, 0)),
                pl.BlockSpec((1, 1, TE), lambda i: (i + goff, 0, 0)),
                pl.BlockSpec((H, H), lambda i: (0, 0)),
                pl.BlockSpec((H, H), lambda i: (0, 0)),
                pl.BlockSpec((H,), lambda i: (0,)),
            ],
            out_specs=pl.BlockSpec((TE, H), lambda i: (i, 0)),
            out_shape=jax.ShapeDtypeStruct((E_c, H), jnp.float32),
        )(P_c, init_edge_states, nd3, Wi1b, Wi2, bi2)

    def sc_scatter(co, NB, imp_c):
        EPW = NB * K
        KS = 40
        NBs = EPW // KS
        assert NBs * KS == EPW
        return pl.kernel(
            functools.partial(_sc_scatter_body, KS, NBs, EPW, NPT, co, H),
            out_type=jax.ShapeDtypeStruct((_NC, N_pad, H), jnp.float32),
            mesh=mesh,
            compiler_params=sc_params,
            scratch_types=(
                [pltpu.VMEM_SHARED((N_pad, H), jnp.float32)]
                + [pltpu.VMEM((KS,), jnp.int32)] * 4
                + [pltpu.VMEM((KS, H), jnp.float32)] * 4
                + [pltpu.VMEM((NPT // 5, H), jnp.float32)]
                + [pltpu.SemaphoreType.DMA] * 8
            ),
        )(imp_c, G, src, dst)

    ps_list = []
    pc_list = []
    co = 0
    for c in range(CH):
        NB_c = nbs[c]
        P_c, pcnt_c = sc_gather(co, NB_c)
        imp_c = tc_edge(co, NB_c, P_c)
        ps_list.append(sc_scatter(co, NB_c, imp_c))
        pc_list.append(pcnt_c)
        co += NB_c * K * _NW

    whole = lambda a: pl.BlockSpec(a.shape, lambda i: (0,) * a.ndim)
    pad_spec = [pl.BlockSpec((_NC, N, H), lambda i: (0, 0, 0))] * CH \
        + [pl.BlockSpec((_NC, N, 16), lambda i: (0, 0, 0))] * CH
    small = [m_t, Wu1, bu1, Wu2, bu2, gamma, beta]
    out = pl.pallas_call(
        functools.partial(_final_body, CH),
        grid=(1,),
        in_specs=pad_spec + [whole(a) for a in small],
        out_specs=pl.BlockSpec((N, H), lambda i: (0, 0)),
        out_shape=jax.ShapeDtypeStruct((N, H), jnp.float32),
    )(*ps_list, *pc_list, *small)
    return out


# speedup vs baseline: 1.0415x; 1.0415x over previous
import functools
import math

import jax
import jax.numpy as jnp
from jax import lax
from jax.experimental import pallas as pl
from jax.experimental.pallas import tpu as pltpu
from jax.experimental.pallas import tpu_sc as plsc

_NC = 2
_NS = 16
_NW = _NC * _NS


def _silu(v):
    return v * jax.nn.sigmoid(v)


def _node_body(x_ref, wi1a_ref, wi1c_ref, bi1_ref, wm1_ref, bm1_ref,
               wm2_ref, bm2_ref, m_ref, a_ref, c_ref, g_ref, mt_ref):
    x = x_ref[...]
    a_ref[...] = (jnp.dot(x, wi1a_ref[...], preferred_element_type=jnp.float32)
                  + bi1_ref[...][None, :])
    c_ref[...] = jnp.dot(x, wi1c_ref[...], preferred_element_type=jnp.float32)
    t = _silu(jnp.dot(x, wm1_ref[...], preferred_element_type=jnp.float32)
              + bm1_ref[...][None, :])
    g = _silu(jnp.dot(t, wm2_ref[...], preferred_element_type=jnp.float32)
              + bm2_ref[...][None, :])
    g_ref[...] = g
    mt_ref[...] = jnp.dot(g, m_ref[...], preferred_element_type=jnp.float32)


def _edge_body(p_ref, ies_ref, nd_ref, wi1b_ref, wi2_ref, bi2_ref, out_ref):
    pre = p_ref[...] + jnp.dot(ies_ref[...], wi1b_ref[...],
                               preferred_element_type=jnp.float32)
    h = jnp.dot(_silu(pre), wi2_ref[...], preferred_element_type=jnp.float32)
    h = h + bi2_ref[...][None, :]
    decay = jnp.cos((math.pi / 2.0) * nd_ref[0, 0, :])
    out_ref[...] = decay[:, None] * _silu(h)


def _final_body(CH, *refs):
    ps = refs[0:CH]
    pc = refs[CH:2 * CH]
    (mt_ref, wu1_ref, bu1_ref, wu2_ref, bu2_ref,
     gamma_ref, beta_ref, out_ref) = refs[2 * CH:]
    sums = ps[0][0] + ps[0][1]
    cnt = pc[0][0, :, 0:1] + pc[0][1, :, 0:1]
    for r in ps[1:]:
        sums = sums + r[0] + r[1]
    for r in pc[1:]:
        cnt = cnt + r[0, :, 0:1] + r[1, :, 0:1]
    incoming = sums / jnp.maximum(cnt, 1.0)
    z = mt_ref[...] + incoming
    h = _silu(jnp.dot(z, wu1_ref[...], preferred_element_type=jnp.float32)
              + bu1_ref[...][None, :])
    h = (jnp.dot(h, wu2_ref[...], preferred_element_type=jnp.float32)
         + bu2_ref[...][None, :])
    mean = jnp.mean(h, axis=0, keepdims=True)
    var = jnp.mean((h - mean) ** 2, axis=0, keepdims=True)
    h = (h - mean) / jnp.sqrt(var + 1e-5) * gamma_ref[...][None, :] \
        + beta_ref[...][None, :]
    out_ref[...] = _silu(h)


def _sc_gather_body(K, NB, EPW, NPT, CO, FIRST, *refs):
    if FIRST:
        (a_hbm, c_hbm, src_hbm, dst_hbm, p_hbm, cnt_out, cnt_sh,
         idx_s0, idx_d0, idx_s1, idx_d1, ra0, rc0, ra1, rc1, rw,
         zcnt, ones_v,
         sis0, sid0, sis1, sid1, sga0, sgc0, sga1, sgc1, swr) = refs
        prev_cnt = None
    else:
        (a_hbm, c_hbm, src_hbm, dst_hbm, prev_cnt, p_hbm, cnt_out, cnt_sh,
         idx_s0, idx_d0, idx_s1, idx_d1, ra0, rc0, ra1, rc1, rw,
         zcnt, ones_v,
         sis0, sid0, sis1, sid1, sga0, sgc0, sga1, sgc1, swr) = refs
    cid = lax.axis_index("c")
    sid = lax.axis_index("s")
    wid = sid * _NC + cid
    base = wid * EPW
    row0 = sid * NPT
    nch = ra0.shape[1] // 16
    zero16 = jnp.zeros((16,), jnp.float32)
    one16 = jnp.ones((16,), jnp.float32)

    def onesfill(i, carry):
        ones_v[i, :] = one16
        return carry

    lax.fori_loop(0, K, onesfill, 0)

    if FIRST:
        def zcfill(i, carry):
            zcnt[i, :] = zero16
            return carry

        lax.fori_loop(0, NPT, zcfill, 0)
        pltpu.sync_copy(zcnt, cnt_sh.at[pl.ds(row0, NPT)])
    else:
        pltpu.sync_copy(prev_cnt.at[cid, pl.ds(row0, NPT)],
                        cnt_sh.at[pl.ds(row0, NPT)])
    plsc.subcore_barrier()

    I = [(idx_s0, idx_d0, sis0, sid0), (idx_s1, idx_d1, sis1, sid1)]
    S = [(ra0, rc0, sga0, sgc0), (ra1, rc1, sga1, sgc1)]

    def issue_idx(jn, p):
        ibs, ibd, ss, sd = I[p]
        off = CO + base + jn * K
        pltpu.async_copy(src_hbm.at[pl.ds(off, K)], ibs, ss)
        pltpu.async_copy(dst_hbm.at[pl.ds(off, K)], ibd, sd)

    def wait_idx(p):
        ibs, ibd, ss, sd = I[p]
        pltpu.make_async_copy(src_hbm.at[pl.ds(0, K)], ibs, ss).wait()
        pltpu.make_async_copy(dst_hbm.at[pl.ds(0, K)], ibd, sd).wait()

    def issue_gathers(p):
        ibs, ibd = I[p][0], I[p][1]
        ra, rc, sga, sgc = S[p]
        pltpu.async_copy(a_hbm.at[ibs], ra, sga)
        pltpu.async_copy(c_hbm.at[ibd], rc, sgc)

    def wait_gathers(p):
        ibs = I[p][0]
        ra, rc, sga, sgc = S[p]
        pltpu.make_async_copy(a_hbm.at[ibs], ra, sga).wait()
        pltpu.make_async_copy(c_hbm.at[ibs], rc, sgc).wait()

    def drain_write():
        pltpu.make_async_copy(rw, p_hbm.at[pl.ds(0, K)], swr).wait()

    def finish(j, p, first):
        ra, rc = S[p][0], S[p][1]
        if not first:
            drain_write()

        @plsc.parallel_loop(0, K, 1, unroll=4)
        def addrow(i):
            for kk in range(nch):
                sl = pl.ds(kk * 16, 16)
                rw[i, sl] = ra[i, sl] + rc[i, sl]

        pltpu.async_copy(rw, p_hbm.at[pl.ds(base + j * K, K)], swr)
        pltpu.sync_copy(ones_v, cnt_sh.at[I[p][1]], add=True)

    def body(j, p, first=False):
        wait_gathers(p)
        wait_idx(1 - p)
        issue_gathers(1 - p)
        finish(j, p, first)
        issue_idx(jnp.minimum(j + 2, NB - 1), p)

    issue_idx(0, 0)
    issue_idx(1, 1)
    wait_idx(0)
    issue_gathers(0)
    body(jnp.int32(0), 0, first=True)

    def pair(j2, carry):
        j = 2 * j2 + 1
        body(j, 1)
        body(j + 1, 0)
        return carry

    lax.fori_loop(0, (NB - 2) // 2, pair, 0)
    if (NB - 2) % 2 == 1:
        body(jnp.int32(NB - 2), 1)
    pe = (NB - 1) % 2
    wait_gathers(pe)
    finish(jnp.int32(NB - 1), pe, False)
    wait_idx(NB % 2)
    drain_write()
    plsc.subcore_barrier()
    pltpu.sync_copy(cnt_sh.at[pl.ds(row0, NPT)],
                    cnt_out.at[cid, pl.ds(row0, NPT)])


def _sc_scatter_body(K, NB, EPW, NPT, CO, H, FIRST, *refs):
    if FIRST:
        (imp_hbm, g_hbm, src_hbm, dst_hbm, sums_out, sums_sh,
         idx_s0, idx_d0, idx_s1, idx_d1, m_v0, g_v0, m_v1, g_v1, zrow,
         sis0, sid0, sis1, sid1, sga0, smv0, sga1, smv1) = refs
        prev_sums = None
    else:
        (imp_hbm, g_hbm, src_hbm, dst_hbm, prev_sums, sums_out, sums_sh,
         idx_s0, idx_d0, idx_s1, idx_d1, m_v0, g_v0, m_v1, g_v1, zrow,
         sis0, sid0, sis1, sid1, sga0, smv0, sga1, smv1) = refs
    cid = lax.axis_index("c")
    sid = lax.axis_index("s")
    wid = sid * _NC + cid
    base = wid * EPW
    row0 = sid * NPT
    nch = H // 16
    zero16 = jnp.zeros((16,), jnp.float32)

    zr = NPT // 5

    if FIRST:
        def zfill(i, carry):
            for kk in range(nch):
                zrow[i, pl.ds(kk * 16, 16)] = zero16
            return carry

        lax.fori_loop(0, zr, zfill, 0)
        for q in range(5):
            pltpu.sync_copy(zrow, sums_sh.at[pl.ds(row0 + q * zr, zr)])
    else:
        pltpu.sync_copy(prev_sums.at[cid, pl.ds(row0, NPT)],
                        sums_sh.at[pl.ds(row0, NPT)])
    plsc.subcore_barrier()

    I = [(idx_s0, idx_d0, sis0, sid0), (idx_s1, idx_d1, sis1, sid1)]
    S = [(m_v0, g_v0, sga0, smv0), (m_v1, g_v1, sga1, smv1)]

    def issue_idx(jn, p):
        ibs, ibd, ss, sd = I[p]
        off = CO + base + jn * K
        pltpu.async_copy(src_hbm.at[pl.ds(off, K)], ibs, ss)
        pltpu.async_copy(dst_hbm.at[pl.ds(off, K)], ibd, sd)

    def wait_idx(p):
        ibs, ibd, ss, sd = I[p]
        pltpu.make_async_copy(src_hbm.at[pl.ds(0, K)], ibs, ss).wait()
        pltpu.make_async_copy(dst_hbm.at[pl.ds(0, K)], ibd, sd).wait()

    def issue_gathers(j, p):
        ibs = I[p][0]
        mv, gv, sga, smv = S[p]
        pltpu.async_copy(g_hbm.at[ibs], gv, sga)
        pltpu.async_copy(imp_hbm.at[pl.ds(base + j * K, K)], mv, smv)

    def wait_gathers(p):
        ibs = I[p][0]
        mv, gv, sga, smv = S[p]
        pltpu.make_async_copy(g_hbm.at[ibs], gv, sga).wait()
        pltpu.make_async_copy(imp_hbm.at[pl.ds(0, K)], mv, smv).wait()

    def finish(p):
        mv, gv = S[p][0], S[p][1]

        @plsc.parallel_loop(0, K, 1, unroll=4)
        def mulrow(i):
            for kk in range(nch):
                sl = pl.ds(kk * 16, 16)
                mv[i, sl] = mv[i, sl] * gv[i, sl]

        pltpu.sync_copy(mv, sums_sh.at[I[p][1]], add=True)

    def body(j, p):
        wait_gathers(p)
        wait_idx(1 - p)
        issue_gathers(j + 1, 1 - p)
        finish(p)
        issue_idx(jnp.minimum(j + 2, NB - 1), p)

    issue_idx(0, 0)
    issue_idx(1, 1)
    wait_idx(0)
    issue_gathers(jnp.int32(0), 0)

    def pair(j2, carry):
        j = 2 * j2
        body(j, 0)
        body(j + 1, 1)
        return carry

    lax.fori_loop(0, (NB - 1) // 2, pair, 0)
    if (NB - 1) % 2 == 1:
        body(jnp.int32(NB - 2), 0)
    pe = (NB - 1) % 2
    wait_gathers(pe)
    finish(pe)
    wait_idx(NB % 2)
    plsc.subcore_barrier()
    pltpu.sync_copy(sums_sh.at[pl.ds(row0, NPT)],
                    sums_out.at[cid, pl.ds(row0, NPT)])


def kernel(x, edge_index, norm_distance, init_edge_states, Wi1, bi1, Wi2, bi2,
           Wm1, bm1, Wm2, bm2, Wu1, bu1, Wu2, bu2, gamma, beta, M):
    N, H = x.shape
    E = edge_index.shape[1]
    src = edge_index[0]
    dst = edge_index[1]
    Wi1a = Wi1[:H]
    Wi1b = Wi1[H:2 * H]
    Wi1c = Wi1[2 * H:]

    A, C, G, m_t = pl.pallas_call(
        _node_body,
        out_shape=[jax.ShapeDtypeStruct((N, H), jnp.float32)] * 4,
    )(x, Wi1a, Wi1c, bi1, Wm1, bm1, Wm2, bm2, M)

    CH = 3
    K = 80
    EPW_total = E // _NW
    NB_total = EPW_total // K
    assert EPW_total * _NW == E and NB_total * K == EPW_total
    assert K % 8 == 0 and K <= 128
    nb0 = NB_total // CH
    nbs = [nb0] * (CH - 1) + [NB_total - nb0 * (CH - 1)]
    NPT = 640
    N_pad = _NS * NPT
    assert N <= N_pad and NPT % 8 == 0 and NPT % 5 == 0
    mesh = plsc.VectorSubcoreMesh(core_axis_name="c", subcore_axis_name="s")
    sc_params = pltpu.CompilerParams(use_tc_tiling_on_sc=False)

    TE = 640
    assert (E // TE) * TE == E
    nd3 = norm_distance.reshape(E // TE, 1, TE)

    def sc_gather(co, NB, prev_cnt):
        EPW = NB * K
        E_c = EPW * _NW
        first = prev_cnt is None
        ins = (A, C, src, dst) if first else (A, C, src, dst, prev_cnt)
        return pl.kernel(
            functools.partial(_sc_gather_body, K, NB, EPW, NPT, co, first),
            out_type=[jax.ShapeDtypeStruct((E_c, H), jnp.float32),
                      jax.ShapeDtypeStruct((_NC, N_pad, 16), jnp.float32)],
            mesh=mesh,
            compiler_params=sc_params,
            scratch_types=(
                [pltpu.VMEM_SHARED((N_pad, 16), jnp.float32)]
                + [pltpu.VMEM((K,), jnp.int32)] * 4
                + [pltpu.VMEM((K, H), jnp.float32)] * 5
                + [pltpu.VMEM((NPT, 16), jnp.float32),
                   pltpu.VMEM((K, 16), jnp.float32)]
                + [pltpu.SemaphoreType.DMA] * 9
            ),
        )(*ins)

    def tc_edge(co, NB, P_c):
        E_c = NB * K * _NW
        GE_c = E_c // TE
        assert GE_c * TE == E_c
        goff = co // TE
        return pl.pallas_call(
            _edge_body,
            grid=(GE_c,),
            in_specs=[
                pl.BlockSpec((TE, H), lambda i: (i, 0)),
                pl.BlockSpec((TE, H), lambda i: (i + goff, 0)),
                pl.BlockSpec((1, 1, TE), lambda i: (i + goff, 0, 0)),
                pl.BlockSpec((H, H), lambda i: (0, 0)),
                pl.BlockSpec((H, H), lambda i: (0, 0)),
                pl.BlockSpec((H,), lambda i: (0,)),
            ],
            out_specs=pl.BlockSpec((TE, H), lambda i: (i, 0)),
            out_shape=jax.ShapeDtypeStruct((E_c, H), jnp.float32),
        )(P_c, init_edge_states, nd3, Wi1b, Wi2, bi2)

    def sc_scatter(co, NB, imp_c, prev_sums):
        EPW = NB * K
        KS = 40
        NBs = EPW // KS
        assert NBs * KS == EPW
        first = prev_sums is None
        ins = ((imp_c, G, src, dst) if first
               else (imp_c, G, src, dst, prev_sums))
        return pl.kernel(
            functools.partial(_sc_scatter_body, KS, NBs, EPW, NPT, co, H,
                              first),
            out_type=jax.ShapeDtypeStruct((_NC, N_pad, H), jnp.float32),
            mesh=mesh,
            compiler_params=sc_params,
            scratch_types=(
                [pltpu.VMEM_SHARED((N_pad, H), jnp.float32)]
                + [pltpu.VMEM((KS,), jnp.int32)] * 4
                + [pltpu.VMEM((KS, H), jnp.float32)] * 4
                + [pltpu.VMEM((NPT // 5, H), jnp.float32)]
                + [pltpu.SemaphoreType.DMA] * 8
            ),
        )(*ins)

    psums = None
    pcnt = None
    co = 0
    for c in range(CH):
        NB_c = nbs[c]
        P_c, pcnt = sc_gather(co, NB_c, pcnt)
        imp_c = tc_edge(co, NB_c, P_c)
        psums = sc_scatter(co, NB_c, imp_c, psums)
        co += NB_c * K * _NW

    whole = lambda a: pl.BlockSpec(a.shape, lambda i: (0,) * a.ndim)
    pad_spec = [pl.BlockSpec((_NC, N, H), lambda i: (0, 0, 0)),
                pl.BlockSpec((_NC, N, 16), lambda i: (0, 0, 0))]
    small = [m_t, Wu1, bu1, Wu2, bu2, gamma, beta]
    out = pl.pallas_call(
        functools.partial(_final_body, 1),
        grid=(1,),
        in_specs=pad_spec + [whole(a) for a in small],
        out_specs=pl.BlockSpec((N, H), lambda i: (0, 0)),
        out_shape=jax.ShapeDtypeStruct((N, H), jnp.float32),
    )(psums, pcnt, *small)
    return out
